# baseline (speedup 1.0000x reference)
import functools

import jax
import jax.numpy as jnp
from jax.experimental import pallas as pl
from jax.experimental.pallas import tpu as pltpu

EPS = 1e-6
TQ = 256


def _knn_kernel(gate_ref, q_ref, km_ref, vm_ref, o_ref, out_ref, kn_ref,
                *, B, S, H, k):
    h = pl.program_id(0)
    qt = pl.program_id(1)

    @pl.when(qt == 0)
    def _():
        km = km_ref[0, :, :]
        kn_ref[...] = km * jax.lax.rsqrt(
            jnp.maximum(jnp.sum(km * km, axis=1, keepdims=True), EPS * EPS))

    q = q_ref[0, :, :]
    kn = kn_ref[...]
    vm = vm_ref[0, :, :]

    qn = q * jax.lax.rsqrt(
        jnp.maximum(jnp.sum(q * q, axis=1, keepdims=True), EPS * EPS))

    sims = jax.lax.dot_general(
        qn, kn, (((1,), (1,)), ((), ())),
        preferred_element_type=jnp.float32)

    neg_inf = jnp.float32(-jnp.inf)
    NS = 4
    CH = 128
    TQr = sims.shape[0]
    nch = sims.shape[1] // CH
    npass = (k + NS - 1) // NS
    t = None
    done = 0
    for p in range(npass):
        accs = [jnp.full((TQr, CH), neg_inf, jnp.float32)
                for _ in range(NS)]
        for c in range(nch):
            x = sims[:, c * CH:(c + 1) * CH]
            if p > 0:
                x = jnp.where(x < t, x, neg_inf)
            for j in range(NS):
                hi = jnp.maximum(accs[j], x)
                x = jnp.minimum(accs[j], x)
                accs[j] = hi
        pool = jnp.concatenate(accs, axis=1)
        r = min(NS, k - done)
        tt = jnp.max(pool, axis=1, keepdims=True)
        for _ in range(r - 1):
            tt = jnp.max(jnp.where(pool < tt, pool, neg_inf), axis=1,
                         keepdims=True)
        t = tt
        done += r
    w = jnp.where(sims >= t, sims, jnp.float32(0.0))

    wm = jax.lax.dot_general(
        w, vm, (((1,), (0,)), ((), ())),
        preferred_element_type=jnp.float32)

    h_eff = jax.lax.rem(h * B + (qt * TQ) // S, H)
    g_raw = gate_ref[h_eff, 0]
    g = jnp.float32(1.0) / (jnp.float32(1.0) + jnp.exp(-g_raw))
    out_ref[0, :, :] = g * wm + (1.0 - g) * o_ref[0, :, :]


@jax.jit
def _run(query, outputs, gate, key_memories, value_memories):
    B, H, S, D = query.shape
    M = key_memories.shape[1]
    Q = B * S
    k = min(H, M)

    q = jnp.transpose(query, (1, 0, 2, 3)).reshape(H, Q, D)
    o = outputs.reshape(H, Q, D)
    gate2 = gate.reshape(H, 1)

    grid = (H, Q // TQ)
    combined = pl.pallas_call(
        functools.partial(_knn_kernel, B=B, S=S, H=H, k=k),
        grid=grid,
        in_specs=[
            pl.BlockSpec(memory_space=pltpu.SMEM),
            pl.BlockSpec((1, TQ, D), lambda h, qt: (h, qt, 0)),
            pl.BlockSpec((1, M, D), lambda h, qt: (h, 0, 0)),
            pl.BlockSpec((1, M, D), lambda h, qt: (h, 0, 0)),
            pl.BlockSpec((1, TQ, D), lambda h, qt: (h, qt, 0)),
        ],
        out_specs=pl.BlockSpec((1, TQ, D), lambda h, qt: (h, qt, 0)),
        out_shape=jax.ShapeDtypeStruct((H, Q, D), jnp.float32),
        scratch_shapes=[pltpu.VMEM((M, D), jnp.float32)],
        compiler_params=pltpu.CompilerParams(
            dimension_semantics=("arbitrary", "arbitrary")),
    )(gate2, q, key_memories, value_memories, o)

    return combined.reshape(B, H, S, D)


def kernel(inputs, query, key, value, outputs, gate, key_memories, value_memories):
    combined = _run(query, outputs, gate, key_memories, value_memories)
    return (combined, jnp.float32(0.0))


# speedup vs baseline: 1.0506x; 1.0506x over previous
import functools

import jax
import jax.numpy as jnp
from jax.experimental import pallas as pl
from jax.experimental.pallas import tpu as pltpu

EPS = 1e-6
TQ = 512


def _knn_kernel(gate_ref, q_ref, km_ref, vm_ref, o_ref, out_ref, kn_ref,
                *, B, S, H, k):
    h = pl.program_id(0)
    qt = pl.program_id(1)

    @pl.when(qt == 0)
    def _():
        km = km_ref[0, :, :]
        kn_ref[...] = km * jax.lax.rsqrt(
            jnp.maximum(jnp.sum(km * km, axis=1, keepdims=True), EPS * EPS))

    q = q_ref[0, :, :]
    kn = kn_ref[...]
    vm = vm_ref[0, :, :]

    qn = q * jax.lax.rsqrt(
        jnp.maximum(jnp.sum(q * q, axis=1, keepdims=True), EPS * EPS))

    sims = jax.lax.dot_general(
        qn, kn, (((1,), (1,)), ((), ())),
        preferred_element_type=jnp.float32)

    neg_inf = jnp.float32(-jnp.inf)
    NS = 4
    CH = 128
    RG = 128
    TQr = sims.shape[0]
    nch = sims.shape[1] // CH
    npass = (k + NS - 1) // NS
    tgs = []
    for g in range(TQr // RG):
        sg = sims[g * RG:(g + 1) * RG, :]
        t = None
        done = 0
        for p in range(npass):
            accs = [jnp.full((RG, CH), neg_inf, jnp.float32)
                    for _ in range(NS)]
            for c in range(nch):
                x = sg[:, c * CH:(c + 1) * CH]
                if p > 0:
                    x = jnp.where(x < t, x, neg_inf)
                for j in range(NS):
                    hi = jnp.maximum(accs[j], x)
                    x = jnp.minimum(accs[j], x)
                    accs[j] = hi
            pool = jnp.concatenate(accs, axis=1)
            r = min(NS, k - done)
            tt = jnp.max(pool, axis=1, keepdims=True)
            for _ in range(r - 1):
                tt = jnp.max(jnp.where(pool < tt, pool, neg_inf), axis=1,
                             keepdims=True)
            t = tt
            done += r
        tgs.append(t)
    t = jnp.concatenate(tgs, axis=0)
    w = jnp.where(sims >= t, sims, jnp.float32(0.0))

    wm = jax.lax.dot_general(
        w, vm, (((1,), (0,)), ((), ())),
        preferred_element_type=jnp.float32)

    h_eff = jax.lax.rem(h * B + (qt * TQ) // S, H)
    g_raw = gate_ref[h_eff, 0]
    g = jnp.float32(1.0) / (jnp.float32(1.0) + jnp.exp(-g_raw))
    out_ref[0, :, :] = g * wm + (1.0 - g) * o_ref[0, :, :]


@jax.jit
def _run(query, outputs, gate, key_memories, value_memories):
    B, H, S, D = query.shape
    M = key_memories.shape[1]
    Q = B * S
    k = min(H, M)

    q = jnp.transpose(query, (1, 0, 2, 3)).reshape(H, Q, D)
    o = outputs.reshape(H, Q, D)
    gate2 = gate.reshape(H, 1)

    grid = (H, Q // TQ)
    combined = pl.pallas_call(
        functools.partial(_knn_kernel, B=B, S=S, H=H, k=k),
        grid=grid,
        in_specs=[
            pl.BlockSpec(memory_space=pltpu.SMEM),
            pl.BlockSpec((1, TQ, D), lambda h, qt: (h, qt, 0)),
            pl.BlockSpec((1, M, D), lambda h, qt: (h, 0, 0)),
            pl.BlockSpec((1, M, D), lambda h, qt: (h, 0, 0)),
            pl.BlockSpec((1, TQ, D), lambda h, qt: (h, qt, 0)),
        ],
        out_specs=pl.BlockSpec((1, TQ, D), lambda h, qt: (h, qt, 0)),
        out_shape=jax.ShapeDtypeStruct((H, Q, D), jnp.float32),
        scratch_shapes=[pltpu.VMEM((M, D), jnp.float32)],
        compiler_params=pltpu.CompilerParams(
            dimension_semantics=("arbitrary", "arbitrary")),
    )(gate2, q, key_memories, value_memories, o)

    return combined.reshape(B, H, S, D)


def kernel(inputs, query, key, value, outputs, gate, key_memories, value_memories):
    combined = _run(query, outputs, gate, key_memories, value_memories)
    return (combined, jnp.float32(0.0))


# speedup vs baseline: 1.1404x; 1.0855x over previous
import functools

import jax
import jax.numpy as jnp
from jax.experimental import pallas as pl
from jax.experimental.pallas import tpu as pltpu

EPS = 1e-6
TQ = 512


def _knn_kernel(gate_ref, q_ref, km_ref, vm_ref, o_ref, out_ref, kn_ref,
                *, B, S, H, k):
    h = pl.program_id(0)
    qt = pl.program_id(1)

    @pl.when(qt == 0)
    def _():
        km = km_ref[0, :, :]
        kn_ref[...] = km * jax.lax.rsqrt(
            jnp.maximum(jnp.sum(km * km, axis=1, keepdims=True), EPS * EPS))

    q = q_ref[0, :, :]
    kn = kn_ref[...]
    vm = vm_ref[0, :, :]

    qn = q * jax.lax.rsqrt(
        jnp.maximum(jnp.sum(q * q, axis=1, keepdims=True), EPS * EPS))

    sims = jax.lax.dot_general(
        qn, kn, (((1,), (1,)), ((), ())),
        preferred_element_type=jnp.float32)

    neg_inf = jnp.float32(-jnp.inf)
    NS = 4
    CH = 128
    TQr = sims.shape[0]
    nch = sims.shape[1] // CH
    npass = (k + NS - 1) // NS
    t = None
    done = 0
    for p in range(npass):
        accs = [jnp.full((TQr, CH), neg_inf, jnp.float32)
                for _ in range(NS)]
        for c4 in range(nch // 4):
            xs = [sims[:, (4 * c4 + i) * CH:(4 * c4 + i + 1) * CH]
                  for i in range(4)]
            if p > 0:
                xs = [jnp.where(x < t, x, neg_inf) for x in xs]
            h1 = jnp.maximum(xs[0], xs[1]); l1 = jnp.minimum(xs[0], xs[1])
            h2 = jnp.maximum(xs[2], xs[3]); l2 = jnp.minimum(xs[2], xs[3])
            b1 = jnp.maximum(h1, h2); m1 = jnp.minimum(h1, h2)
            m2 = jnp.maximum(l1, l2); b4 = jnp.minimum(l1, l2)
            b2 = jnp.maximum(m1, m2); b3 = jnp.minimum(m1, m2)
            a1, a2, a3, a4 = accs
            c1 = jnp.maximum(a1, b1); u = jnp.minimum(a1, b1)
            v = jnp.maximum(a3, b3)
            c2 = jnp.maximum(u, v); c3 = jnp.minimum(u, v)
            d1 = jnp.maximum(a2, b2); u2 = jnp.minimum(a2, b2)
            v2 = jnp.maximum(a4, b4)
            d2 = jnp.maximum(u2, v2)
            accs = [c1,
                    jnp.maximum(d1, c2), jnp.minimum(d1, c2),
                    jnp.maximum(d2, c3)]
        pool = jnp.concatenate(accs, axis=1)
        r = min(NS, k - done)
        tt = jnp.max(pool, axis=1, keepdims=True)
        for _ in range(r - 1):
            tt = jnp.max(jnp.where(pool < tt, pool, neg_inf), axis=1,
                         keepdims=True)
        t = tt
        done += r
    w = jnp.where(sims >= t, sims, jnp.float32(0.0))

    wm = jax.lax.dot_general(
        w, vm, (((1,), (0,)), ((), ())),
        preferred_element_type=jnp.float32)

    h_eff = jax.lax.rem(h * B + (qt * TQ) // S, H)
    g_raw = gate_ref[h_eff, 0]
    g = jnp.float32(1.0) / (jnp.float32(1.0) + jnp.exp(-g_raw))
    out_ref[0, :, :] = g * wm + (1.0 - g) * o_ref[0, :, :]


@jax.jit
def _run(query, outputs, gate, key_memories, value_memories):
    B, H, S, D = query.shape
    M = key_memories.shape[1]
    Q = B * S
    k = min(H, M)

    q = jnp.transpose(query, (1, 0, 2, 3)).reshape(H, Q, D)
    o = outputs.reshape(H, Q, D)
    gate2 = gate.reshape(H, 1)

    grid = (H, Q // TQ)
    combined = pl.pallas_call(
        functools.partial(_knn_kernel, B=B, S=S, H=H, k=k),
        grid=grid,
        in_specs=[
            pl.BlockSpec(memory_space=pltpu.SMEM),
            pl.BlockSpec((1, TQ, D), lambda h, qt: (h, qt, 0)),
            pl.BlockSpec((1, M, D), lambda h, qt: (h, 0, 0)),
            pl.BlockSpec((1, M, D), lambda h, qt: (h, 0, 0)),
            pl.BlockSpec((1, TQ, D), lambda h, qt: (h, qt, 0)),
        ],
        out_specs=pl.BlockSpec((1, TQ, D), lambda h, qt: (h, qt, 0)),
        out_shape=jax.ShapeDtypeStruct((H, Q, D), jnp.float32),
        scratch_shapes=[pltpu.VMEM((M, D), jnp.float32)],
        compiler_params=pltpu.CompilerParams(
            dimension_semantics=("arbitrary", "arbitrary")),
    )(gate2, q, key_memories, value_memories, o)

    return combined.reshape(B, H, S, D)


def kernel(inputs, query, key, value, outputs, gate, key_memories, value_memories):
    combined = _run(query, outputs, gate, key_memories, value_memories)
    return (combined, jnp.float32(0.0))
